# Initial kernel scaffold; baseline (speedup 1.0000x reference)
import jax
import jax.numpy as jnp
from jax.experimental import pallas as pl


def kernel(adj_lit, adj_clause, clauses_mask_sigmoid, clauses_mask_softplus, clause_graph_ids, variable_graph_ids, params):
    raise NotImplementedError("write your pallas kernel here")


# speedup vs baseline: 3.7283x; 3.7283x over previous
import functools

import jax
import jax.numpy as jnp
from jax import lax
from jax.experimental import pallas as pl
from jax.experimental.pallas import tpu as pltpu
from jax.experimental.pallas import tpu_sc as plsc

N_V = 10000
N_C = 40000
L2 = 2 * N_V
E = 120000
G = 64
FM = 64
QM = 64
ROUNDS = 4

CH = 128
TILES = 16
EPT = 7680
EPAD = TILES * EPT
NCH = EPT // CH
RB = 2000

f32 = jnp.float32
i32 = jnp.int32


def _make_edge_pass(n_src, n_dst, w, ept=EPT):
    pt = n_dst // TILES
    ptz = ((pt + CH - 1) // CH) * CH
    acc_rows = TILES * ptz
    nch = ept // CH

    def shp(*dims):
        return dims if w > 1 else dims[:-1]

    out_sds = jax.ShapeDtypeStruct(shp(acc_rows, w), f32)

    @functools.cache
    def build():
        mesh = plsc.VectorSubcoreMesh(core_axis_name="c", subcore_axis_name="s")
        return functools.partial(
            pl.kernel,
            out_type=(out_sds, out_sds),
            mesh=mesh,
            scratch_types=[
                pltpu.VMEM((CH,), i32),
                pltpu.VMEM((CH,), i32),
                pltpu.VMEM(shp(CH, w), f32),
                pltpu.VMEM_SHARED(shp(acc_rows, w), f32),
                pltpu.SemaphoreType.DMA,
            ],
            compiler_params=pltpu.CompilerParams(use_tc_tiling_on_sc=False),
        )(kf)

    def kf(src0, src1, ia, ib, z, out0, out1, iav, ibv, rows, acc, sem):
        c = lax.axis_index("c")
        s = lax.axis_index("s")

        def work(src, out):
            pltpu.sync_copy(z, acc.at[pl.ds(s * ptz, ptz)])
            plsc.subcore_barrier()

            def body(i, carry):
                off = s * ept + i * CH
                pltpu.sync_copy(ia.at[pl.ds(off, CH)], iav)
                pltpu.async_copy(src.at[iav], rows, sem).wait()
                pltpu.sync_copy(ib.at[pl.ds(off, CH)], ibv)
                pltpu.sync_copy(rows, acc.at[ibv], add=True)
                return carry

            lax.fori_loop(0, nch, body, 0)
            plsc.subcore_barrier()
            pltpu.sync_copy(acc.at[pl.ds(s * ptz, ptz)],
                            out.at[pl.ds(s * ptz, ptz)])

        @pl.when(c == 0)
        def _():
            work(src0, out0)

        @pl.when(c == 1)
        def _():
            work(src1, out1)

    def run(src0, src1, ia, ib):
        z = jnp.zeros(shp(ptz, w), f32)
        o0, o1 = build()(src0, src1, ia, ib, z)
        return o0[:n_dst], o1[:n_dst]

    return run


EPT2 = 10240
E2 = TILES * EPT2


def _make_gather(n_src, w, ept=EPT2):
    nch = ept // CH
    out_sds = jax.ShapeDtypeStruct((TILES * ept, w), f32)

    @functools.cache
    def build():
        mesh = plsc.VectorSubcoreMesh(core_axis_name="c", subcore_axis_name="s")
        return functools.partial(
            pl.kernel,
            out_type=(out_sds, out_sds),
            mesh=mesh,
            scratch_types=[
                pltpu.VMEM((CH,), i32),
                pltpu.VMEM((CH, w), f32),
                pltpu.SemaphoreType.DMA,
            ],
            compiler_params=pltpu.CompilerParams(use_tc_tiling_on_sc=False),
        )(kf)

    def kf(src0, src1, ia, out0, out1, iav, rows, sem):
        c = lax.axis_index("c")
        s = lax.axis_index("s")

        def work(src, out):
            def body(i, carry):
                off = s * ept + i * CH
                pltpu.sync_copy(ia.at[pl.ds(off, CH)], iav)
                pltpu.async_copy(src.at[iav], rows, sem).wait()
                pltpu.sync_copy(rows, out.at[pl.ds(off, CH)])
                return carry

            lax.fori_loop(0, nch, body, 0)

        @pl.when(c == 0)
        def _():
            work(src0, out0)

        @pl.when(c == 1)
        def _():
            work(src1, out1)

    return lambda src0, src1, ia: build()(src0, src1, ia)


def _make_scatter(n_dst, w, ept=EPT2):
    pt = n_dst // TILES
    ptz = ((pt + CH - 1) // CH) * CH
    acc_rows = TILES * ptz
    nch = ept // CH
    out_sds = jax.ShapeDtypeStruct((acc_rows, w), f32)

    @functools.cache
    def build():
        mesh = plsc.VectorSubcoreMesh(core_axis_name="c", subcore_axis_name="s")
        return functools.partial(
            pl.kernel,
            out_type=(out_sds, out_sds),
            mesh=mesh,
            scratch_types=[
                pltpu.VMEM((CH,), i32),
                pltpu.VMEM((CH, w), f32),
                pltpu.VMEM_SHARED((acc_rows, w), f32),
                pltpu.SemaphoreType.DMA,
            ],
            compiler_params=pltpu.CompilerParams(use_tc_tiling_on_sc=False),
        )(kf)

    def kf(src0, src1, ib, z, out0, out1, ibv, rows, acc, sem):
        c = lax.axis_index("c")
        s = lax.axis_index("s")

        def work(src, out):
            pltpu.sync_copy(z, acc.at[pl.ds(s * ptz, ptz)])
            plsc.subcore_barrier()

            def body(i, carry):
                off = s * ept + i * CH
                pltpu.sync_copy(src.at[pl.ds(off, CH)], rows)
                pltpu.sync_copy(ib.at[pl.ds(off, CH)], ibv)
                pltpu.sync_copy(rows, acc.at[ibv], add=True)
                return carry

            lax.fori_loop(0, nch, body, 0)
            plsc.subcore_barrier()
            pltpu.sync_copy(acc.at[pl.ds(s * ptz, ptz)],
                            out.at[pl.ds(s * ptz, ptz)])

        @pl.when(c == 0)
        def _():
            work(src0, out0)

        @pl.when(c == 1)
        def _():
            work(src1, out1)

    def run(src0, src1, ib):
        z = jnp.zeros((ptz, w), f32)
        o0, o1 = build()(src0, src1, ib, z)
        return o0[:n_dst], o1[:n_dst]

    return run


_pass_p0 = _make_edge_pass(N_C, L2, 16)
_pass_a = _make_edge_pass(L2, N_C, 32)
_pass_b = _make_edge_pass(N_C, L2, 64)
_pass_c = _make_edge_pass(L2, N_C, 16)
_gather_a = _make_gather(L2, 32)
_gather_ev = _make_gather(N_C, 16)
_scatter_a = _make_scatter(N_C, 32)


def _dot(a, b):
    return jnp.dot(a, b, preferred_element_type=f32)


def _segdot(onehot, v):
    return lax.dot_general(onehot, v, (((0,), (0,)), ((), ())),
                           preferred_element_type=f32,
                           precision=lax.Precision.HIGHEST)


def _onehot(gid_blk):
    return (gid_blk == lax.broadcasted_iota(i32, (1, G), 1)).astype(f32)


def _softplus(x):
    return jnp.maximum(x, 0.0) + jnp.log1p(jnp.exp(-jnp.abs(x)))


def _kexp_probe(x, out):
    out[...] = jnp.exp(-x[...])


def _blk(d):
    return pl.BlockSpec((RB, d), lambda g: (g, 0))


def _whole(r, c_):
    return pl.BlockSpec((r, c_), lambda g: (0, 0))


def _call(body, nrows, in_specs, out_specs, out_shapes):
    return pl.pallas_call(
        body,
        grid=(nrows // RB,),
        in_specs=in_specs,
        out_specs=out_specs,
        out_shape=out_shapes,
    )


def _kpre_a(cms, cmsp, cgid, p0lo, cgacc):
    @pl.when(pl.program_id(0) == 0)
    def _():
        cgacc[...] = jnp.zeros_like(cgacc)

    p0lo[...] = cms[...] * cmsp[...]
    cgacc[...] += _segdot(_onehot(cgid[...]), cms[...])


def _kwnorm(w, gsum, gid, out):
    g = gsum[...]
    inv = jnp.where(g > 0, 1.0 / jnp.where(g > 0, g, 1.0), 0.0)
    out[...] = w[...] * _dot(_onehot(gid[...]), inv)


def _kpre_c(mp, mn, lp, ln, vgid, wvar, vdw, vgacc):
    @pl.when(pl.program_id(0) == 0)
    def _():
        vgacc[...] = jnp.zeros_like(vgacc)

    wv = 1.0 - jnp.exp(-(mp[...] + mn[...]))
    wvar[...] = wv
    vdw[...] = 4.0 * lax.rsqrt(jnp.maximum(lp[...] + ln[...], 1.0))
    vgacc[...] += _segdot(_onehot(vgid[...]), wv)


def _kdw(ld, dw):
    dw[...] = lax.rsqrt(jnp.maximum(ld[...], 1.0))


def _ksoftplus1(x, out):
    out[...] = _softplus(x[...])


def _kmlp2(x, w1, b1, w2, b2, out):
    h = jnp.maximum(_dot(x[...], w1[...]) + b1[...], 0.0)
    out[...] = _dot(h, w2[...]) + b2[...]


def _ksptable(lits, sp0, sp1):
    sp = _softplus(lits[...])
    sp0[...] = sp[:, :32]
    sp1[...] = sp[:, 32:]


def _kclause(cs, t0, t1, cms, cgid, wcn, w1, b1, w2, b2,
             pb0, pb1, x, meanacc):
    @pl.when(pl.program_id(0) == 0)
    def _():
        meanacc[...] = jnp.zeros_like(meanacc)

    t = jnp.concatenate([t0[...], t1[...]], axis=1)
    c = jnp.exp(-cms[...] * t)
    u = jnp.concatenate([cs[...], 4.0 * c], axis=1)
    h = jnp.maximum(_dot(u, w1[...]) + b1[...], 0.0)
    data = _dot(h, w2[...]) + b2[...]
    xb = data[:, QM:]
    pb0[...] = cms[...] * c
    pb1[...] = cms[...] * data[:, :QM]
    x[...] = xb
    meanacc[...] += _segdot(_onehot(cgid[...]), xb * wcn[...])


def _kpnapply(x, mean, wn, gid, y, msqacc):
    @pl.when(pl.program_id(0) == 0)
    def _():
        msqacc[...] = jnp.zeros_like(msqacc)

    oh = _onehot(gid[...])
    yb = x[...] - _dot(oh, mean[...])
    y[...] = yb
    msqacc[...] += _segdot(oh, wn[...] * jnp.mean(yb * yb, axis=1, keepdims=True))


def _kpnfinal(y, msq, gid, old, out):
    r = lax.rsqrt(msq[...] + 1e-6)
    out[...] = y[...] * _dot(_onehot(gid[...]), r) * 0.25 + 0.1 * old[...]


RB2 = 2048


def _blk2(d):
    return pl.BlockSpec((RB2, d), lambda g: (g, 0))


def _kmulev(r0, r1, ev, o0, o1):
    e = ev[...][:, :1]
    o0[...] = r0[...] * e
    o1[...] = r1[...] * e


def _kublock(q, sp_, sn_, vp, vn, dwp, dwn, vdw, vr, vgid, wvn, spp, spn,
             w1, b1, w2, b2, w3, b3, uo, meanacc):
    @pl.when(pl.program_id(0) == 0)
    def _():
        meanacc[...] = jnp.zeros_like(meanacc)

    dp = jnp.exp(q[...] - spp[...])
    dn = jnp.exp(-q[...] - spn[...])
    grad = ((-sp_[...]) * dp - (-sn_[...]) * dn) * vdw[...]
    unit = jnp.concatenate(
        [grad, vr[...], vp[...] * dwp[...], vn[...] * dwn[...]], axis=1)
    h = jnp.maximum(_dot(unit, w1[...]) + b1[...], 0.0)
    h = jnp.maximum(_dot(h, w2[...]) + b2[...], 0.0)
    uob = _dot(h, w3[...]) + b3[...]
    uo[...] = uob
    meanacc[...] += _segdot(_onehot(vgid[...]), uob * wvn[...])


def _kloss(u, cms, cgid, pgacc):
    @pl.when(pl.program_id(0) == 0)
    def _():
        pgacc[...] = jnp.zeros_like(pgacc)

    cv = jnp.exp(-cms[...] * u[...])
    pcl = cv * (-jnp.log(jnp.maximum(1.0 - cv, 0.0) + 1e-8))
    pgacc[...] += _segdot(_onehot(cgid[...]), cms[...] * pcl)


def _sds(*s):
    return jax.ShapeDtypeStruct(s, f32)


_SC_ON = dict(p0=True, a=True, b=True, c=True)
_PURE_JNP = False


def _jnp_pipeline(adj_lit, adj_clause, cms, cmsp, cgid, vgid, params):
    import numpy as np

    def _mlpr(x, layers):
        for (W, b) in layers[:-1]:
            x = jax.nn.relu(x @ W + b)
        W, b = layers[-1]
        return x @ W + b

    def _pair_norm(x, w_norm, gid, num_graphs):
        mean = jax.ops.segment_sum(x * w_norm[:, None], gid, num_segments=num_graphs)
        x = x - mean[gid]
        msq = jax.ops.segment_sum(w_norm * jnp.mean(x * x, axis=1), gid, num_segments=num_graphs)
        return x * jax.lax.rsqrt(msq + 1e-6)[gid][:, None]

    edge_val = cms[adj_clause]
    w_clause = cms
    masked = edge_val * cmsp[adj_clause]
    var_sum = jax.ops.segment_sum(masked, adj_lit, num_segments=2 * N_V)
    w_var = 1.0 - jnp.exp(-(var_sum[:N_V] + var_sum[N_V:]))
    lit_degree = jax.ops.segment_sum(edge_val, adj_lit, num_segments=2 * N_V)[:, None]
    degree_weight = jax.lax.rsqrt(jnp.maximum(lit_degree, 1.0))
    var_degree_weight = 4.0 * jax.lax.rsqrt(jnp.maximum(lit_degree[:N_V] + lit_degree[N_V:], 1.0))
    cg_sum = jax.ops.segment_sum(w_clause, cgid, num_segments=G)
    cg_inv = jnp.where(cg_sum > 0, 1.0 / jnp.where(cg_sum > 0, cg_sum, 1.0), 0.0)
    w_clause_norm = w_clause * cg_inv[cgid]
    vg_sum = jax.ops.segment_sum(w_var, vgid, num_segments=G)
    vg_inv = jnp.where(vg_sum > 0, 1.0 / jnp.where(vg_sum > 0, vg_sum, 1.0), 0.0)
    w_var_norm = w_var * vg_inv[vgid]

    def _clause_vals(q):
        lits = jnp.concatenate([q, -q], axis=0)
        sp = jax.nn.softplus(lits)
        agg = jax.ops.segment_sum(sp[adj_lit] * edge_val[:, None], adj_clause, num_segments=N_C)
        return jnp.exp(-agg)

    def _clause_vals_mine(q):
        lits = jnp.concatenate([q, -q], axis=0)
        sp = jax.nn.softplus(lits)
        t = jax.ops.segment_sum(sp[adj_lit], adj_clause, num_segments=N_C)
        return jnp.exp(-cms[:, None] * t)

    variables = jnp.ones((N_V, FM), dtype=jnp.float32)
    clause_state = jnp.ones((N_C, FM), dtype=jnp.float32)
    nkey = jax.random.key(42)
    loss_acc = jnp.zeros((G, 1), dtype=jnp.float32)
    last_logits = jnp.zeros((N_V, 1), dtype=jnp.float32)
    for step in range(ROUNDS):
        noise = jax.random.normal(jax.random.fold_in(nkey, step), (N_V, 4), dtype=jnp.float32)
        v1 = jnp.concatenate([variables, noise], axis=-1)
        query = _mlpr(v1, params['q'])
        clauses_loss = _clause_vals(query)
        variables_grad = jax.grad(lambda q: jnp.sum(_clause_vals(q)))(query)
        variables_grad = variables_grad * var_degree_weight
        clauses_loss = clauses_loss * 4.0
        clause_unit = jnp.concatenate([clause_state, clauses_loss], axis=-1)
        clause_data = _mlpr(clause_unit, params['c'])
        variables_loss_all = clause_data[:, :QM]
        new_clause = _pair_norm(clause_data[:, QM:], w_clause_norm, cgid, G) * 0.25
        clause_state = new_clause + 0.1 * clause_state
        vloss = jax.ops.segment_sum(edge_val[:, None] * variables_loss_all[adj_clause], adj_lit, num_segments=2 * N_V)
        vloss = vloss * degree_weight
        unit = jnp.concatenate([variables_grad, variables, vloss[:N_V], vloss[N_V:]], axis=-1)
        new_vars = _pair_norm(_mlpr(unit, params['u']), w_var_norm, vgid, G) * 0.25
        variables = new_vars + 0.1 * variables
        logits = _mlpr(variables, params['o'])
        cv = _clause_vals(logits)
        per_clause = cv * (-jnp.log(1.0 - cv + 1e-8))
        per_graph = jax.ops.segment_sum(w_clause[:, None] * per_clause, cgid, num_segments=G)
        per_graph = jnp.sqrt(per_graph + 1e-6) - np.sqrt(1e-6)
        loss_acc = loss_acc + per_graph
        last_logits = logits
        variables = jax.lax.stop_gradient(variables) * 0.2 + variables * 0.8
        clause_state = jax.lax.stop_gradient(clause_state) * 0.2 + clause_state * 0.8
    return last_logits, loss_acc / float(ROUNDS)


def _jnp_pipeline_mine(adj_lit, adj_clause, cms, cmsp, cgid, vgid, params):
    seg = lambda v, idx, n: jax.ops.segment_sum(v, idx, num_segments=n)
    msum = seg((cms * cmsp)[adj_clause], adj_lit, L2)
    ldeg = seg(cms[adj_clause], adj_lit, L2)
    w_var = 1.0 - jnp.exp(-(msum[:N_V] + msum[N_V:]))
    dw = lax.rsqrt(jnp.maximum(ldeg, 1.0))[:, None]
    vdw = 4.0 * lax.rsqrt(jnp.maximum(ldeg[:N_V] + ldeg[N_V:], 1.0))[:, None]
    cg = seg(cms, cgid, G)
    cg_inv = jnp.where(cg > 0, 1.0 / jnp.where(cg > 0, cg, 1.0), 0.0)
    wcn = cms * cg_inv[cgid]
    vg = seg(w_var, vgid, G)
    vg_inv = jnp.where(vg > 0, 1.0 / jnp.where(vg > 0, vg, 1.0), 0.0)
    wvn = w_var * vg_inv[vgid]

    def pair_norm(x, w, gid):
        mean = seg(x * w[:, None], gid, G)
        x = x - mean[gid]
        msq = seg(w * jnp.mean(x * x, axis=1), gid, G)
        return x * lax.rsqrt(msq + 1e-6)[gid][:, None]

    def mlp(x, layers):
        for (W, b) in layers[:-1]:
            x = jax.nn.relu(_dot(x, W) + b)
        W, b = layers[-1]
        return _dot(x, W) + b

    variables = jnp.ones((N_V, FM), f32)
    clause_state = jnp.ones((N_C, FM), f32)
    nkey = jax.random.key(42)
    loss_acc = jnp.zeros((G, 1), f32)
    for step in range(ROUNDS):
        noise = jax.random.normal(jax.random.fold_in(nkey, step), (N_V, 4), f32)
        q = mlp(jnp.concatenate([variables, noise], -1), params['q'])
        sp = jax.nn.softplus(jnp.concatenate([q, -q], 0))
        T = seg(sp[adj_lit], adj_clause, N_C)
        c = jnp.exp(-cms[:, None] * T)
        unit_c = jnp.concatenate([clause_state, 4.0 * c], -1)
        data = mlp(unit_c, params['c'])
        VL = data[:, :QM]
        new_c = pair_norm(data[:, QM:], wcn, cgid) * 0.25
        clause_state = new_c + 0.1 * clause_state
        S = seg((cms[:, None] * c)[adj_clause], adj_lit, L2)
        vpre = seg((cms[:, None] * VL)[adj_clause], adj_lit, L2)
        grad = (-jax.nn.sigmoid(q) * S[:N_V] + jax.nn.sigmoid(-q) * S[N_V:]) * vdw
        vloss = vpre * dw
        unit = jnp.concatenate([grad, variables, vloss[:N_V], vloss[N_V:]], -1)
        nv = pair_norm(mlp(unit, params['u']), wvn, vgid) * 0.25
        variables = nv + 0.1 * variables
        logits = mlp(variables, params['o'])
        lsp = jax.nn.softplus(jnp.concatenate([logits, -logits], 0)[:, 0])
        U = seg(lsp[adj_lit], adj_clause, N_C)
        cv = jnp.exp(-cms * U)[:, None]
        pc = cv * (-jnp.log(1.0 - cv + 1e-8))
        pg = seg(cms[:, None] * pc, cgid, G)
        loss_acc = loss_acc + jnp.sqrt(pg + 1e-6) - jnp.sqrt(jnp.float32(1e-6))
    return logits, loss_acc / float(ROUNDS)


def kernel(adj_lit, adj_clause, clauses_mask_sigmoid, clauses_mask_softplus,
           clause_graph_ids, variable_graph_ids, params):
    if _PURE_JNP:
        return _jnp_pipeline(adj_lit, adj_clause, clauses_mask_sigmoid,
                             clauses_mask_softplus, clause_graph_ids,
                             variable_graph_ids, params)
    def _sc(flag, passfn, s0, s1, ia, ib, ra, rb, n_dst):
        if _SC_ON[flag]:
            return passfn(s0, s1, ia, ib)
        return (jax.ops.segment_sum(s0[ra], rb, num_segments=n_dst),
                jax.ops.segment_sum(s1[ra], rb, num_segments=n_dst))
    cms = clauses_mask_sigmoid.reshape(N_C, 1)
    cmsp = clauses_mask_softplus.reshape(N_C, 1)
    cgid = clause_graph_ids.reshape(N_C, 1)
    vgid = variable_graph_ids.reshape(N_V, 1)

    npad = EPAD - E
    ia_lit = jnp.concatenate([adj_lit, jnp.full((npad,), L2, i32)])
    ia_cl = jnp.concatenate([adj_clause, jnp.full((npad,), N_C, i32)])
    ib_lit = jnp.concatenate([adj_lit, jnp.zeros((npad,), i32)])
    ib_cl = jnp.concatenate([adj_clause, jnp.zeros((npad,), i32)])

    def _edge_lists(ia_r, ib_r, ptz, pad_src):
        tile = ib_r // ptz
        order = jnp.argsort(tile, stable=True)
        ia_s, ib_s, t_s = ia_r[order], ib_r[order], tile[order]
        cnt = jnp.zeros((TILES,), i32).at[tile].add(1)
        start = jnp.concatenate(
            [jnp.zeros((1,), i32), jnp.cumsum(cnt)[:-1].astype(i32)])
        pos = jnp.arange(E, dtype=i32) - start[t_s]
        slot = t_s * EPT2 + pos
        ia2 = jnp.full((E2,), pad_src, i32).at[slot].set(ia_s)
        base = ((jnp.arange(E2, dtype=i32) // EPT2) * ptz).astype(i32)
        ib2 = base.at[slot].set(ib_s)
        return ia2, ib2

    iaA2, ibA2 = _edge_lists(adj_lit, adj_clause, 2560, L2)
    iaB2, ibB2 = _edge_lists(adj_clause, adj_lit, 1280, N_C)

    def padsrc(x, n):
        if x.ndim == 1:
            return jnp.concatenate([x, jnp.zeros((8,), f32)])
        return jnp.concatenate([x, jnp.zeros((8, x.shape[1]), f32)], axis=0)

    qp, cp, up, op = params['q'], params['c'], params['u'], params['o']
    qW1, qb1 = qp[0][0], qp[0][1].reshape(1, -1)
    qW2, qb2 = qp[1][0], qp[1][1].reshape(1, -1)
    cW1, cb1 = cp[0][0], cp[0][1].reshape(1, -1)
    cW2, cb2 = cp[1][0], cp[1][1].reshape(1, -1)
    uW1, ub1 = up[0][0], up[0][1].reshape(1, -1)
    uW2, ub2 = up[1][0], up[1][1].reshape(1, -1)
    uW3, ub3 = up[2][0], up[2][1].reshape(1, -1)
    oW1, ob1 = op[0][0], op[0][1].reshape(1, -1)
    oW2, ob2 = op[1][0], op[1][1].reshape(1, -1)

    p0lo, cg_sum = _call(
        _kpre_a, N_C,
        [_blk(1), _blk(1), _blk(1)],
        [_blk(1), _whole(G, 1)],
        (_sds(N_C, 1), _sds(G, 1)),
    )(cms, cmsp, cgid)

    wcn = _call(
        _kwnorm, N_C,
        [_blk(1), _whole(G, 1), _blk(1)],
        _blk(1), _sds(N_C, 1),
    )(cms, cg_sum, cgid)

    def pad16(col):
        return jnp.pad(col, ((0, 8), (0, 15)))

    msum16, ldeg16 = _sc('p0', _pass_p0, pad16(p0lo), pad16(cms), ia_cl, ib_lit,
                         adj_clause, adj_lit, L2)
    msum = msum16[:, :1]
    ldeg = ldeg16[:, :1]

    wvar, vdw, vg_sum = _call(
        _kpre_c, N_V,
        [_blk(1)] * 5,
        [_blk(1), _blk(1), _whole(G, 1)],
        (_sds(N_V, 1), _sds(N_V, 1), _sds(G, 1)),
    )(msum[:N_V], msum[N_V:], ldeg[:N_V], ldeg[N_V:], vgid)

    wvn = _call(
        _kwnorm, N_V,
        [_blk(1), _whole(G, 1), _blk(1)],
        _blk(1), _sds(N_V, 1),
    )(wvar, vg_sum, vgid)

    dw = _call(_kdw, L2, [_blk(1)], _blk(1), _sds(L2, 1))(ldeg)

    variables = jnp.ones((N_V, FM), f32)
    clause_state = jnp.ones((N_C, FM), f32)
    nkey = jax.random.key(42)
    loss_acc = jnp.zeros((G, 1), f32)
    logits = jnp.zeros((N_V, 1), f32)

    for step in range(ROUNDS):
        noise = jax.random.normal(jax.random.fold_in(nkey, step), (N_V, 4), f32)
        xq = jnp.concatenate([variables, noise], axis=1)
        q = _call(
            _kmlp2, N_V,
            [_blk(FM + 4), _whole(FM + 4, QM), _whole(1, QM),
             _whole(QM, QM), _whole(1, QM)],
            _blk(QM), _sds(N_V, QM),
        )(xq, qW1, qb1, qW2, qb2)

        lits = jnp.concatenate([q, -q], axis=0)
        sp0, sp1 = _call(
            _ksptable, L2, [_blk(QM)],
            [_blk(32), _blk(32)],
            (_sds(L2, 32), _sds(L2, 32)),
        )(lits)

        t0, t1 = _sc('a', _pass_a, padsrc(sp0, 0), padsrc(sp1, 0), ia_lit,
                     ib_cl, adj_lit, adj_clause, N_C)

        pb0, pb1, xc, mean_c = _call(
            _kclause, N_C,
            [_blk(FM), _blk(32), _blk(32), _blk(1), _blk(1), _blk(1),
             _whole(FM + QM, 2 * FM), _whole(1, 2 * FM),
             _whole(2 * FM, FM + QM), _whole(1, FM + QM)],
            [_blk(FM), _blk(FM), _blk(FM), _whole(G, FM)],
            (_sds(N_C, FM), _sds(N_C, FM), _sds(N_C, FM), _sds(G, FM)),
        )(clause_state, t0, t1, cms, cgid, wcn, cW1, cb1, cW2, cb2)

        yc, msq_c = _call(
            _kpnapply, N_C,
            [_blk(FM), _whole(G, FM), _blk(1), _blk(1)],
            [_blk(FM), _whole(G, 1)],
            (_sds(N_C, FM), _sds(G, 1)),
        )(xc, mean_c, wcn, cgid)

        clause_state = _call(
            _kpnfinal, N_C,
            [_blk(FM), _whole(G, 1), _blk(1), _blk(FM)],
            _blk(FM), _sds(N_C, FM),
        )(yc, msq_c, cgid, clause_state)

        sres, vres = _sc('b', _pass_b, padsrc(pb0, 0), padsrc(pb1, 0), ia_cl,
                         ib_lit, adj_clause, adj_lit, L2)

        spp = jnp.concatenate([sp0[:N_V], sp1[:N_V]], axis=1)
        spn = jnp.concatenate([sp0[N_V:], sp1[N_V:]], axis=1)
        uo, mean_v = _call(
            _kublock, N_V,
            [_blk(QM), _blk(QM), _blk(QM), _blk(FM), _blk(FM),
             _blk(1), _blk(1), _blk(1), _blk(FM), _blk(1), _blk(1),
             _blk(FM), _blk(FM),
             _whole(4 * FM, 2 * FM), _whole(1, 2 * FM),
             _whole(2 * FM, 2 * FM), _whole(1, 2 * FM),
             _whole(2 * FM, FM), _whole(1, FM)],
            [_blk(FM), _whole(G, FM)],
            (_sds(N_V, FM), _sds(G, FM)),
        )(q, sres[:N_V], sres[N_V:], vres[:N_V], vres[N_V:],
          dw[:N_V], dw[N_V:], vdw, variables, vgid, wvn, spp, spn,
          uW1, ub1, uW2, ub2, uW3, ub3)

        yv, msq_v = _call(
            _kpnapply, N_V,
            [_blk(FM), _whole(G, FM), _blk(1), _blk(1)],
            [_blk(FM), _whole(G, 1)],
            (_sds(N_V, FM), _sds(G, 1)),
        )(uo, mean_v, wvn, vgid)

        variables = _call(
            _kpnfinal, N_V,
            [_blk(FM), _whole(G, 1), _blk(1), _blk(FM)],
            _blk(FM), _sds(N_V, FM),
        )(yv, msq_v, vgid, variables)

        logits = _call(
            _kmlp2, N_V,
            [_blk(FM), _whole(FM, FM), _whole(1, FM),
             _whole(FM, 1), _whole(1, 1)],
            _blk(1), _sds(N_V, 1),
        )(variables, oW1, ob1, oW2, ob2)

        litlog = jnp.concatenate([logits, -logits], axis=0)
        lsp = _call(
            _ksoftplus1, L2, [_blk(1)], _blk(1), _sds(L2, 1),
        )(litlog)

        upad = pad16(lsp)
        ucl, _unused = _sc('c', _pass_c, upad, upad, ia_lit, ib_cl,
                           adj_lit, adj_clause, N_C)

        pg = _call(
            _kloss, N_C,
            [_blk(1), _blk(1), _blk(1)],
            _whole(G, 1), _sds(G, 1),
        )(ucl[:, :1], cms, cgid)

        loss_acc = loss_acc + jnp.sqrt(pg + 1e-6) - jnp.sqrt(jnp.float32(1e-6))
        variables = jax.lax.stop_gradient(variables) * 0.2 + variables * 0.8
        clause_state = (jax.lax.stop_gradient(clause_state) * 0.2
                        + clause_state * 0.8)

    return logits, loss_acc / float(ROUNDS)
